# baseline (speedup 1.0000x reference)
import functools

import jax
import jax.numpy as jnp
from jax import lax
from jax.experimental import pallas as pl
from jax.experimental.pallas import tpu as pltpu
from jax.experimental.pallas import tpu_sc as plsc

N = 10000
D = 128
E_RAW = 320000
E_TOT = E_RAW + N
NC, NS, L = 2, 16, 16

EB = 128
NB0, NB1 = 92, 70
NBMAX = max(NB0, NB1)
C0E = NS * NB0 * EB
EP = NS * (NB0 + NB1) * EB
TRASH = N
SROW = 640
RPAD = NS * SROW
ZROWS = 128

_mesh = plsc.VectorSubcoreMesh(
    core_axis_name="c", subcore_axis_name="s", num_cores=NC, num_subcores=NS
)


@functools.partial(
    pl.kernel,
    out_type=jax.ShapeDtypeStruct((NC, RPAD), jnp.float32),
    mesh=_mesh,
    scratch_types=[
        pltpu.VMEM_SHARED((RPAD,), jnp.float32),
        pltpu.VMEM((NBMAX, EB), jnp.int32),
        pltpu.VMEM((EB,), jnp.float32),
        pltpu.VMEM((SROW,), jnp.float32),
        pltpu.SemaphoreType.DMA,
        pltpu.SemaphoreType.DMA,
    ],
)
def _deg_kernel(dst_hbm, out_hbm, acc, slab, ones, zbuf, slabsem, ssem):
    c = lax.axis_index("c")
    s = lax.axis_index("s")
    nb = lax.select(c == 0, NB0, NB1)
    slab_cp = pltpu.async_copy(dst_hbm.at[c * NS + s], slab, slabsem)

    def _fill(i, _):
        zbuf[pl.ds(i * L, L)] = jnp.zeros((L,), jnp.float32)
        return 0

    lax.fori_loop(0, SROW // L, _fill, 0)

    def _fill1(i, _):
        ones[pl.ds(i * L, L)] = jnp.ones((L,), jnp.float32)
        return 0

    lax.fori_loop(0, EB // L, _fill1, 0)

    pltpu.sync_copy(zbuf, acc.at[pl.ds(s * SROW, SROW)])
    slab_cp.wait()
    plsc.subcore_barrier()

    def _body(i, _):
        pltpu.async_copy(ones, acc.at[slab.at[i]], ssem, add=True)
        return 0

    lax.fori_loop(0, nb, _body, 0)

    def _drain(i, _):
        pltpu.make_async_copy(ones, acc.at[slab.at[i]], ssem).wait()
        return 0

    lax.fori_loop(0, nb, _drain, 0)
    plsc.subcore_barrier()
    pltpu.sync_copy(acc.at[pl.ds(s * SROW, SROW)], out_hbm.at[c, pl.ds(s * SROW, SROW)])


@functools.partial(
    pl.kernel,
    out_type=jax.ShapeDtypeStruct((NC, RPAD, D), jnp.float32),
    mesh=_mesh,
    scratch_types=[
        pltpu.VMEM_SHARED((RPAD, D), jnp.float32),
        pltpu.VMEM((NBMAX, EB), jnp.int32),
        pltpu.VMEM((EB,), jnp.int32),
        pltpu.VMEM((EB,), jnp.int32),
        pltpu.VMEM((EB,), jnp.int32),
        pltpu.VMEM((EB,), jnp.int32),
        pltpu.VMEM((EB, D), jnp.float32),
        pltpu.VMEM((EB, D), jnp.float32),
        pltpu.SemaphoreType.DMA,
        pltpu.SemaphoreType.DMA,
        pltpu.SemaphoreType.DMA,
        pltpu.SemaphoreType.DMA,
        pltpu.SemaphoreType.DMA,
    ],
)
def _spmm_kernel(
    pk_hbm, h_hbm, out_hbm,
    acc, slab, idxs0, idxs1, idxd0, idxd1, rows0, rows1,
    gs0, gs1, ss0, ss1, slabsem,
):
    c = lax.axis_index("c")
    s = lax.axis_index("s")
    tile = c * NS + s
    nb = lax.select(c == 0, NB0, NB1)
    slab_cp = pltpu.async_copy(pk_hbm.at[tile], slab, slabsem)

    def _fill(i, _):
        for j in range(D // L):
            rows0[i, pl.ds(j * L, L)] = jnp.zeros((L,), jnp.float32)
        return 0

    lax.fori_loop(0, ZROWS, _fill, 0)
    for k in range(SROW // ZROWS):
        pltpu.sync_copy(rows0, acc.at[pl.ds(s * SROW + k * ZROWS, ZROWS)])
    slab_cp.wait()
    plsc.subcore_barrier()

    idxs = (idxs0, idxs1)
    idxd = (idxd0, idxd1)
    rows = (rows0, rows1)
    gs = (gs0, gs1)
    ss = (ss0, ss1)

    def _unpack(g, buf):
        for j in range(EB // L):
            v = slab[g, pl.ds(j * L, L)]
            idxs[buf][pl.ds(j * L, L)] = lax.shift_right_logical(v, 14)
            idxd[buf][pl.ds(j * L, L)] = lax.bitwise_and(v, 16383)

    _unpack(0, 0)
    pltpu.async_copy(h_hbm.at[idxs0], rows0, gs0)

    def _half(g, cur, nxt):
        @pl.when(g >= 1)
        def _drain_prev():
            pltpu.make_async_copy(rows[nxt], acc.at[idxd[nxt]], ss[nxt]).wait()

        @pl.when(g + 1 < nb)
        def _start_next():
            _unpack(g + 1, nxt)
            pltpu.async_copy(h_hbm.at[idxs[nxt]], rows[nxt], gs[nxt])

        pltpu.make_async_copy(h_hbm.at[idxs[cur]], rows[cur], gs[cur]).wait()
        pltpu.async_copy(rows[cur], acc.at[idxd[cur]], ss[cur], add=True)

    def _body(g, _):
        @pl.when(lax.rem(g, 2) == 0)
        def _even():
            _half(g, 0, 1)

        @pl.when(lax.rem(g, 2) == 1)
        def _odd():
            _half(g, 1, 0)

        return 0

    lax.fori_loop(0, nb, _body, 0)
    pltpu.make_async_copy(rows1, acc.at[idxd1], ss1).wait()
    plsc.subcore_barrier()
    pltpu.sync_copy(acc.at[pl.ds(s * SROW, SROW)], out_hbm.at[c, pl.ds(s * SROW, SROW)])


R = 512
NBLK = RPAD // R


def _dinv_of(degp_ref):
    i = pl.program_id(0)
    deg = degp_ref[0, pl.ds(i * R, R)] + degp_ref[1, pl.ds(i * R, R)]
    return jnp.where(deg > 0, lax.rsqrt(deg), 0.0)


def _tc1_body(degp_ref, x_ref, w_ref, out_ref):
    dinv = _dinv_of(degp_ref)
    h = jnp.dot(x_ref[...], w_ref[...], preferred_element_type=jnp.float32)
    out_ref[...] = h * dinv[:, None]


def _tc2_body(degp_ref, sp_ref, b_ref, w_ref, out_ref):
    dinv = _dinv_of(degp_ref)
    sagg = sp_ref[0] + sp_ref[1]
    o = jnp.maximum(sagg * dinv[:, None] + b_ref[...], 0.0)
    out_ref[...] = (
        jnp.dot(o, w_ref[...], preferred_element_type=jnp.float32) * dinv[:, None]
    )


def _tc3_body(degp_ref, sp_ref, b_ref, wl_ref, bl_ref, out_ref):
    dinv = _dinv_of(degp_ref)
    sagg = sp_ref[0] + sp_ref[1]
    o = jnp.maximum(sagg * dinv[:, None] + b_ref[...], 0.0)
    out_ref[...] = jnp.dot(o, wl_ref[...], preferred_element_type=jnp.float32) + bl_ref[...]


_DEGP_SPEC = pl.BlockSpec((2, RPAD), lambda i: (0, 0))
_SP_SPEC = pl.BlockSpec((2, R, D), lambda i: (0, i, 0))
_PARAMS = pltpu.CompilerParams(dimension_semantics=("parallel",))


def _tc1(degp, x, W1):
    return pl.pallas_call(
        _tc1_body,
        grid=(NBLK,),
        in_specs=[
            _DEGP_SPEC,
            pl.BlockSpec((R, D), lambda i: (i, 0)),
            pl.BlockSpec((D, D), lambda i: (0, 0)),
        ],
        out_specs=pl.BlockSpec((R, D), lambda i: (i, 0)),
        out_shape=jax.ShapeDtypeStruct((N, D), jnp.float32),
        compiler_params=_PARAMS,
    )(degp, x, W1)


def _tc2(degp, sp, b1, W2):
    return pl.pallas_call(
        _tc2_body,
        grid=(NBLK,),
        in_specs=[
            _DEGP_SPEC,
            _SP_SPEC,
            pl.BlockSpec((1, D), lambda i: (0, 0)),
            pl.BlockSpec((D, D), lambda i: (0, 0)),
        ],
        out_specs=pl.BlockSpec((R, D), lambda i: (i, 0)),
        out_shape=jax.ShapeDtypeStruct((N, D), jnp.float32),
        compiler_params=_PARAMS,
    )(degp, sp, b1, W2)


def _tc3(degp, sp, b2, Wl, bl):
    nc = Wl.shape[1]
    return pl.pallas_call(
        _tc3_body,
        grid=(NBLK,),
        in_specs=[
            _DEGP_SPEC,
            _SP_SPEC,
            pl.BlockSpec((1, D), lambda i: (0, 0)),
            pl.BlockSpec((D, nc), lambda i: (0, 0)),
            pl.BlockSpec((1, nc), lambda i: (0, 0)),
        ],
        out_specs=pl.BlockSpec((R, nc), lambda i: (i, 0)),
        out_shape=jax.ShapeDtypeStruct((N, nc), jnp.float32),
        compiler_params=_PARAMS,
    )(degp, sp, b2, Wl, bl)


def kernel(x, edge_index, W1, b1, W2, b2, Wl, bl):
    ei = edge_index.astype(jnp.int32)
    loop = jnp.arange(N, dtype=jnp.int32)
    npad = EP - E_TOT
    src = jnp.concatenate([ei[0], loop, jnp.zeros((npad,), jnp.int32)])
    dst = jnp.concatenate([ei[1], loop, jnp.full((npad,), TRASH, jnp.int32)])

    def _slab3(a):
        a0 = a[:C0E].reshape(NS, NB0, EB)
        a0 = jnp.pad(a0, ((0, 0), (0, NBMAX - NB0), (0, 0)))
        a1 = a[C0E:].reshape(NS, NB1, EB)
        a1 = jnp.pad(a1, ((0, 0), (0, NBMAX - NB1), (0, 0)))
        return jnp.concatenate([a0, a1], axis=0)

    pk3 = _slab3(src * 16384 + dst)
    dst3 = _slab3(dst)

    degp = _deg_kernel(dst3)
    h1 = _tc1(degp, x, W1)
    s1 = _spmm_kernel(pk3, h1)
    h2 = _tc2(degp, s1, b1.reshape(1, D), W2)
    s2 = _spmm_kernel(pk3, h2)
    return _tc3(degp, s2, b2.reshape(1, D), Wl, bl.reshape(1, -1))


# speedup vs baseline: 1.1135x; 1.1135x over previous
import functools

import jax
import jax.numpy as jnp
from jax import lax
from jax.experimental import pallas as pl
from jax.experimental.pallas import tpu as pltpu
from jax.experimental.pallas import tpu_sc as plsc

N = 10000
D = 128
E_RAW = 320000
E_TOT = E_RAW + N
NC, NS, L = 2, 16, 16

EB = 128
NB0, NB1 = 100, 62
NBMAX = max(NB0, NB1)
C0E = NS * NB0 * EB
EP = NS * (NB0 + NB1) * EB
TRASH = N
SROW = 640
RPAD = NS * SROW
ZROWS = 128

_mesh = plsc.VectorSubcoreMesh(
    core_axis_name="c", subcore_axis_name="s", num_cores=NC, num_subcores=NS
)


@functools.partial(
    pl.kernel,
    out_type=jax.ShapeDtypeStruct((NC, RPAD), jnp.float32),
    mesh=_mesh,
    scratch_types=[
        pltpu.VMEM_SHARED((RPAD,), jnp.float32),
        pltpu.VMEM((NBMAX, EB), jnp.int32),
        pltpu.VMEM((EB,), jnp.float32),
        pltpu.VMEM((SROW,), jnp.float32),
        pltpu.SemaphoreType.DMA,
        pltpu.SemaphoreType.DMA,
    ],
)
def _deg_kernel(pk_hbm, out_hbm, acc, slab, ones, zbuf, slabsem, ssem):
    c = lax.axis_index("c")
    s = lax.axis_index("s")
    nb = lax.select(c == 0, NB0, NB1)
    slab_cp = pltpu.async_copy(pk_hbm.at[c * NS + s], slab, slabsem)

    def _fill(i, _):
        zbuf[pl.ds(i * L, L)] = jnp.zeros((L,), jnp.float32)
        return 0

    lax.fori_loop(0, SROW // L, _fill, 0)

    def _fill1(i, _):
        ones[pl.ds(i * L, L)] = jnp.ones((L,), jnp.float32)
        return 0

    lax.fori_loop(0, EB // L, _fill1, 0)

    pltpu.sync_copy(zbuf, acc.at[pl.ds(s * SROW, SROW)])
    slab_cp.wait()
    def _mask(i, _):
        for j in range(EB // L):
            slab[i, pl.ds(j * L, L)] = lax.bitwise_and(
                slab[i, pl.ds(j * L, L)], 16383
            )
        return 0

    lax.fori_loop(0, nb, _mask, 0)
    plsc.subcore_barrier()

    def _body(i, _):
        pltpu.async_copy(ones, acc.at[slab.at[i]], ssem, add=True)
        return 0

    lax.fori_loop(0, nb, _body, 0)

    def _drain(i, _):
        pltpu.make_async_copy(ones, acc.at[slab.at[i]], ssem).wait()
        return 0

    lax.fori_loop(0, nb, _drain, 0)
    plsc.subcore_barrier()
    pltpu.sync_copy(acc.at[pl.ds(s * SROW, SROW)], out_hbm.at[c, pl.ds(s * SROW, SROW)])


@functools.partial(
    pl.kernel,
    out_type=jax.ShapeDtypeStruct((NC, RPAD, D), jnp.float32),
    mesh=_mesh,
    scratch_types=[
        pltpu.VMEM_SHARED((RPAD, D), jnp.float32),
        pltpu.VMEM((NBMAX, EB), jnp.int32),
        pltpu.VMEM((EB,), jnp.int32),
        pltpu.VMEM((EB,), jnp.int32),
        pltpu.VMEM((EB,), jnp.int32),
        pltpu.VMEM((EB,), jnp.int32),
        pltpu.VMEM((EB, D), jnp.float32),
        pltpu.VMEM((EB, D), jnp.float32),
        pltpu.SemaphoreType.DMA,
        pltpu.SemaphoreType.DMA,
        pltpu.SemaphoreType.DMA,
        pltpu.SemaphoreType.DMA,
        pltpu.SemaphoreType.DMA,
    ],
)
def _spmm_kernel(
    pk_hbm, h_hbm, out_hbm,
    acc, slab, idxs0, idxs1, idxd0, idxd1, rows0, rows1,
    gs0, gs1, ss0, ss1, slabsem,
):
    c = lax.axis_index("c")
    s = lax.axis_index("s")
    tile = c * NS + s
    nb = lax.select(c == 0, NB0, NB1)
    slab_cp = pltpu.async_copy(pk_hbm.at[tile], slab, slabsem)

    def _fill(i, _):
        for j in range(D // L):
            rows0[i, pl.ds(j * L, L)] = jnp.zeros((L,), jnp.float32)
        return 0

    lax.fori_loop(0, ZROWS, _fill, 0)
    for k in range(SROW // ZROWS):
        pltpu.sync_copy(rows0, acc.at[pl.ds(s * SROW + k * ZROWS, ZROWS)])
    slab_cp.wait()
    plsc.subcore_barrier()

    idxs = (idxs0, idxs1)
    idxd = (idxd0, idxd1)
    rows = (rows0, rows1)
    gs = (gs0, gs1)
    ss = (ss0, ss1)

    def _unpack(g, buf):
        for j in range(EB // L):
            v = slab[g, pl.ds(j * L, L)]
            idxs[buf][pl.ds(j * L, L)] = lax.shift_right_logical(v, 14)
            idxd[buf][pl.ds(j * L, L)] = lax.bitwise_and(v, 16383)

    _unpack(0, 0)
    pltpu.async_copy(h_hbm.at[idxs0], rows0, gs0)

    def _half(g, cur, nxt):
        @pl.when(g >= 1)
        def _drain_prev():
            pltpu.make_async_copy(rows[nxt], acc.at[idxd[nxt]], ss[nxt]).wait()

        @pl.when(g + 1 < nb)
        def _start_next():
            _unpack(g + 1, nxt)
            pltpu.async_copy(h_hbm.at[idxs[nxt]], rows[nxt], gs[nxt])

        pltpu.make_async_copy(h_hbm.at[idxs[cur]], rows[cur], gs[cur]).wait()
        pltpu.async_copy(rows[cur], acc.at[idxd[cur]], ss[cur], add=True)

    def _body(g, _):
        @pl.when(lax.rem(g, 2) == 0)
        def _even():
            _half(g, 0, 1)

        @pl.when(lax.rem(g, 2) == 1)
        def _odd():
            _half(g, 1, 0)

        return 0

    lax.fori_loop(0, nb, _body, 0)
    pltpu.make_async_copy(rows1, acc.at[idxd1], ss1).wait()
    plsc.subcore_barrier()
    pltpu.sync_copy(acc.at[pl.ds(s * SROW, SROW)], out_hbm.at[c, pl.ds(s * SROW, SROW)])


def _dinv_of(degp_ref):
    deg = degp_ref[0, :] + degp_ref[1, :]
    dinv = jnp.where(deg > 0, lax.rsqrt(deg), 0.0)
    return dinv[:N, None]


def _tc1_body(degp_ref, x_ref, w_ref, out_ref):
    h = jnp.dot(x_ref[...], w_ref[...], preferred_element_type=jnp.float32)
    out_ref[...] = h * _dinv_of(degp_ref)


def _tc2_body(degp_ref, sp_ref, b_ref, w_ref, out_ref):
    dinv = _dinv_of(degp_ref)
    sagg = sp_ref[0, :N, :] + sp_ref[1, :N, :]
    o = jnp.maximum(sagg * dinv + b_ref[...], 0.0)
    out_ref[...] = (
        jnp.dot(o, w_ref[...], preferred_element_type=jnp.float32) * dinv
    )


def _tc3_body(degp_ref, sp_ref, b_ref, wl_ref, bl_ref, out_ref):
    dinv = _dinv_of(degp_ref)
    sagg = sp_ref[0, :N, :] + sp_ref[1, :N, :]
    o = jnp.maximum(sagg * dinv + b_ref[...], 0.0)
    out_ref[...] = jnp.dot(o, wl_ref[...], preferred_element_type=jnp.float32) + bl_ref[...]


def _tc1(degp, x, W1):
    return pl.pallas_call(
        _tc1_body,
        out_shape=jax.ShapeDtypeStruct((N, D), jnp.float32),
    )(degp, x, W1)


def _tc2(degp, sp, b1, W2):
    return pl.pallas_call(
        _tc2_body,
        out_shape=jax.ShapeDtypeStruct((N, D), jnp.float32),
    )(degp, sp, b1, W2)


def _tc3(degp, sp, b2, Wl, bl):
    nc = Wl.shape[1]
    return pl.pallas_call(
        _tc3_body,
        out_shape=jax.ShapeDtypeStruct((N, nc), jnp.float32),
    )(degp, sp, b2, Wl, bl)


def kernel(x, edge_index, W1, b1, W2, b2, Wl, bl):
    ei = edge_index.astype(jnp.int32)
    loop = jnp.arange(N, dtype=jnp.int32)
    npad = EP - E_TOT
    pk = jnp.concatenate(
        [
            ei[0] * 16384 + ei[1],
            loop * 16385,
            jnp.full((npad,), TRASH, jnp.int32),
        ]
    )

    a0 = pk[:C0E].reshape(NS, NB0, EB)
    a0 = jnp.pad(a0, ((0, 0), (0, NBMAX - NB0), (0, 0)))
    a1 = pk[C0E:].reshape(NS, NB1, EB)
    a1 = jnp.pad(a1, ((0, 0), (0, NBMAX - NB1), (0, 0)))
    pk3 = jnp.concatenate([a0, a1], axis=0)

    degp = _deg_kernel(pk3)
    h1 = _tc1(degp, x, W1)
    s1 = _spmm_kernel(pk3, h1)
    h2 = _tc2(degp, s1, b1.reshape(1, D), W2)
    s2 = _spmm_kernel(pk3, h2)
    return _tc3(degp, s2, b2.reshape(1, D), Wl, bl.reshape(1, -1))


# speedup vs baseline: 1.1976x; 1.0755x over previous
import functools

import jax
import jax.numpy as jnp
from jax import lax
from jax.experimental import pallas as pl
from jax.experimental.pallas import tpu as pltpu
from jax.experimental.pallas import tpu_sc as plsc

N = 10000
D = 128
E_RAW = 320000
E_TOT = E_RAW + N
NC, NS, L = 2, 16, 16

EB = 128
NB0, NB1 = 100, 62
NBMAX = max(NB0, NB1)
NBS = (NB0 + NB1) // 2
C0E = NS * NB0 * EB
EP = NS * (NB0 + NB1) * EB
TRASH = N
SROW = 640
RPAD = NS * SROW
ZROWS = 128

_mesh = plsc.VectorSubcoreMesh(
    core_axis_name="c", subcore_axis_name="s", num_cores=NC, num_subcores=NS
)


@functools.partial(
    pl.kernel,
    out_type=jax.ShapeDtypeStruct((NC, RPAD), jnp.float32),
    mesh=_mesh,
    scratch_types=[
        pltpu.VMEM_SHARED((RPAD,), jnp.float32),
        pltpu.VMEM((NBS, EB), jnp.int32),
        pltpu.VMEM((EB,), jnp.float32),
        pltpu.VMEM((SROW,), jnp.float32),
        pltpu.SemaphoreType.DMA,
        pltpu.SemaphoreType.DMA,
    ],
)
def _deg_kernel(pk_hbm, out_hbm, acc, slab, ones, zbuf, slabsem, ssem):
    c = lax.axis_index("c")
    s = lax.axis_index("s")
    nb = NBS
    slab_cp = pltpu.async_copy(pk_hbm.at[c * NS + s], slab, slabsem)

    def _fill(i, _):
        zbuf[pl.ds(i * L, L)] = jnp.zeros((L,), jnp.float32)
        return 0

    lax.fori_loop(0, SROW // L, _fill, 0)

    def _fill1(i, _):
        ones[pl.ds(i * L, L)] = jnp.ones((L,), jnp.float32)
        return 0

    lax.fori_loop(0, EB // L, _fill1, 0)

    pltpu.sync_copy(zbuf, acc.at[pl.ds(s * SROW, SROW)])
    slab_cp.wait()
    def _mask(i, _):
        for j in range(EB // L):
            slab[i, pl.ds(j * L, L)] = lax.bitwise_and(
                slab[i, pl.ds(j * L, L)], 16383
            )
        return 0

    lax.fori_loop(0, nb, _mask, 0)
    plsc.subcore_barrier()

    def _body(i, _):
        pltpu.async_copy(ones, acc.at[slab.at[i]], ssem, add=True)
        return 0

    lax.fori_loop(0, nb, _body, 0)

    def _drain(i, _):
        pltpu.make_async_copy(ones, acc.at[slab.at[i]], ssem).wait()
        return 0

    lax.fori_loop(0, nb, _drain, 0)
    plsc.subcore_barrier()
    pltpu.sync_copy(acc.at[pl.ds(s * SROW, SROW)], out_hbm.at[c, pl.ds(s * SROW, SROW)])


@functools.partial(
    pl.kernel,
    out_type=jax.ShapeDtypeStruct((NC, RPAD, D), jnp.float32),
    mesh=_mesh,
    scratch_types=[
        pltpu.VMEM_SHARED((RPAD, D), jnp.float32),
        pltpu.VMEM((NBMAX * EB,), jnp.int32),
        pltpu.VMEM((EB,), jnp.int32),
        pltpu.VMEM((EB,), jnp.int32),
        pltpu.VMEM((EB,), jnp.int32),
        pltpu.VMEM((EB,), jnp.int32),
        pltpu.VMEM((EB, D), jnp.float32),
        pltpu.VMEM((EB, D), jnp.float32),
        pltpu.SemaphoreType.DMA,
        pltpu.SemaphoreType.DMA,
        pltpu.SemaphoreType.DMA,
        pltpu.SemaphoreType.DMA,
        pltpu.SemaphoreType.DMA,
    ],
)
def _spmm_kernel(
    pk_hbm, h_hbm, out_hbm,
    acc, slab, idxs0, idxs1, idxd0, idxd1, rows0, rows1,
    gs0, gs1, ss0, ss1, slabsem,
):
    c = lax.axis_index("c")
    s = lax.axis_index("s")
    nb = lax.select(c == 0, NB0, NB1)

    @pl.when(c == 0)
    def _load0():
        pltpu.async_copy(
            pk_hbm.at[pl.ds(s * (NB0 * EB), NB0 * EB)],
            slab.at[pl.ds(0, NB0 * EB)], slabsem,
        )

    @pl.when(c == 1)
    def _load1():
        pltpu.async_copy(
            pk_hbm.at[pl.ds(C0E + s * (NB1 * EB), NB1 * EB)],
            slab.at[pl.ds(0, NB1 * EB)], slabsem,
        )

    def _fill(i, _):
        for j in range(D // L):
            rows0[i, pl.ds(j * L, L)] = jnp.zeros((L,), jnp.float32)
        return 0

    lax.fori_loop(0, ZROWS, _fill, 0)
    for k in range(SROW // ZROWS):
        pltpu.sync_copy(rows0, acc.at[pl.ds(s * SROW + k * ZROWS, ZROWS)])

    @pl.when(c == 0)
    def _wait0():
        pltpu.make_async_copy(
            pk_hbm.at[pl.ds(s * (NB0 * EB), NB0 * EB)],
            slab.at[pl.ds(0, NB0 * EB)], slabsem,
        ).wait()

    @pl.when(c == 1)
    def _wait1():
        pltpu.make_async_copy(
            pk_hbm.at[pl.ds(C0E + s * (NB1 * EB), NB1 * EB)],
            slab.at[pl.ds(0, NB1 * EB)], slabsem,
        ).wait()

    plsc.subcore_barrier()

    idxs = (idxs0, idxs1)
    idxd = (idxd0, idxd1)
    rows = (rows0, rows1)
    gs = (gs0, gs1)
    ss = (ss0, ss1)

    def _unpack(g, buf):
        for j in range(EB // L):
            v = slab[pl.ds(g * EB + j * L, L)]
            idxs[buf][pl.ds(j * L, L)] = lax.shift_right_logical(v, 14)
            idxd[buf][pl.ds(j * L, L)] = lax.bitwise_and(v, 16383)

    _unpack(0, 0)
    pltpu.async_copy(h_hbm.at[idxs0], rows0, gs0)

    def _half(g, cur, nxt):
        @pl.when(g >= 1)
        def _drain_prev():
            pltpu.make_async_copy(rows[nxt], acc.at[idxd[nxt]], ss[nxt]).wait()

        @pl.when(g + 1 < nb)
        def _start_next():
            _unpack(g + 1, nxt)
            pltpu.async_copy(h_hbm.at[idxs[nxt]], rows[nxt], gs[nxt])

        pltpu.make_async_copy(h_hbm.at[idxs[cur]], rows[cur], gs[cur]).wait()
        pltpu.async_copy(rows[cur], acc.at[idxd[cur]], ss[cur], add=True)

    def _body(g, _):
        @pl.when(lax.rem(g, 2) == 0)
        def _even():
            _half(g, 0, 1)

        @pl.when(lax.rem(g, 2) == 1)
        def _odd():
            _half(g, 1, 0)

        return 0

    lax.fori_loop(0, nb, _body, 0)
    pltpu.make_async_copy(rows1, acc.at[idxd1], ss1).wait()
    plsc.subcore_barrier()
    pltpu.sync_copy(acc.at[pl.ds(s * SROW, SROW)], out_hbm.at[c, pl.ds(s * SROW, SROW)])


def _dinv_of(degp_ref):
    deg = degp_ref[0, :] + degp_ref[1, :]
    dinv = jnp.where(deg > 0, lax.rsqrt(deg), 0.0)
    return dinv[:N, None]


def _tc1_body(degp_ref, x_ref, w_ref, out_ref):
    h = jnp.dot(x_ref[...], w_ref[...], preferred_element_type=jnp.float32)
    out_ref[...] = h * _dinv_of(degp_ref)


def _tc2_body(degp_ref, sp_ref, b_ref, w_ref, out_ref):
    dinv = _dinv_of(degp_ref)
    sagg = sp_ref[0, :N, :] + sp_ref[1, :N, :]
    o = jnp.maximum(sagg * dinv + b_ref[...], 0.0)
    out_ref[...] = (
        jnp.dot(o, w_ref[...], preferred_element_type=jnp.float32) * dinv
    )


def _tc3_body(degp_ref, sp_ref, b_ref, wl_ref, bl_ref, out_ref):
    dinv = _dinv_of(degp_ref)
    sagg = sp_ref[0, :N, :] + sp_ref[1, :N, :]
    o = jnp.maximum(sagg * dinv + b_ref[...], 0.0)
    out_ref[...] = jnp.dot(o, wl_ref[...], preferred_element_type=jnp.float32) + bl_ref[...]


def _tc1(degp, x, W1):
    return pl.pallas_call(
        _tc1_body,
        out_shape=jax.ShapeDtypeStruct((N, D), jnp.float32),
    )(degp, x, W1)


def _tc2(degp, sp, b1, W2):
    return pl.pallas_call(
        _tc2_body,
        out_shape=jax.ShapeDtypeStruct((N, D), jnp.float32),
    )(degp, sp, b1, W2)


def _tc3(degp, sp, b2, Wl, bl):
    nc = Wl.shape[1]
    return pl.pallas_call(
        _tc3_body,
        out_shape=jax.ShapeDtypeStruct((N, nc), jnp.float32),
    )(degp, sp, b2, Wl, bl)


def kernel(x, edge_index, W1, b1, W2, b2, Wl, bl):
    ei = edge_index.astype(jnp.int32)
    loop = jnp.arange(N, dtype=jnp.int32)
    npad = EP - E_TOT
    pk = jnp.concatenate(
        [
            ei[0] * 16384 + ei[1],
            loop * 16385,
            jnp.full((npad,), TRASH, jnp.int32),
        ]
    )

    pk3 = pk.reshape(NC * NS, NBS, EB)

    degp = _deg_kernel(pk3)
    h1 = _tc1(degp, x, W1)
    s1 = _spmm_kernel(pk, h1)
    h2 = _tc2(degp, s1, b1.reshape(1, D), W2)
    s2 = _spmm_kernel(pk, h2)
    return _tc3(degp, s2, b2.reshape(1, D), Wl, bl.reshape(1, -1))
